# baseline (speedup 1.0000x reference)
import functools

import jax
import jax.numpy as jnp
from jax import lax
from jax.experimental import pallas as pl
from jax.experimental.pallas import tpu as pltpu
from jax.experimental.pallas import tpu_sc as plsc

B = 16384
F = 64

_info = plsc.get_sparse_core_info()
_NC, _NS, _NL = _info.num_cores, _info.num_subcores, _info.num_lanes
_NW = _NC * _NS
_BPW = B // _NW
_CHUNK = 128
_NCHUNK = _BPW // _CHUNK

_mesh = plsc.VectorSubcoreMesh(core_axis_name="c", subcore_axis_name="s")


_TBLK = 8192
_M = 1000000
_TGRID = (_M + _TBLK - 1) // _TBLK


def _tr_body(embt_ref, eye_ref, out_ref):
    out_ref[...] = lax.dot_general(
        embt_ref[...], eye_ref[...], (((0,), (0,)), ((), ())),
        preferred_element_type=jnp.float32,
        precision=lax.Precision.HIGHEST)


@jax.jit
def _tru(embt):
    eye = jnp.eye(F, dtype=jnp.float32)
    return pl.pallas_call(
        _tr_body,
        grid=(_TGRID,),
        in_specs=[
            pl.BlockSpec((F, _TBLK), lambda i: (0, i)),
            pl.BlockSpec((F, F), lambda i: (0, 0)),
        ],
        out_specs=pl.BlockSpec((_TBLK, F), lambda i: (i, 0)),
        out_shape=jax.ShapeDtypeStruct((_M, F), jnp.float32),
    )(embt, eye)


def _make_gather_v():
    @functools.partial(
        pl.kernel,
        mesh=_mesh,
        out_type=jax.ShapeDtypeStruct((B, F), jnp.float32),
        scratch_types=[
            pltpu.VMEM((_NCHUNK, _CHUNK), jnp.int32),
            pltpu.VMEM((_BPW, F), jnp.float32),
            pltpu.SemaphoreType.DMA,
        ],
        compiler_params=pltpu.CompilerParams(use_tc_tiling_on_sc=False),
    )
    def gather_v(items_hbm, vemb_hbm, v_out, idxc, rows, sem):
        wid = lax.axis_index("s") * _NC + lax.axis_index("c")
        base = wid * _BPW
        for c in range(_NCHUNK):
            pltpu.sync_copy(
                items_hbm.at[pl.ds(base + c * _CHUNK, _CHUNK)], idxc.at[c])
        copies = [
            pltpu.async_copy(
                vemb_hbm.at[idxc.at[c]],
                rows.at[pl.ds(c * _CHUNK, _CHUNK)], sem)
            for c in range(_NCHUNK)
        ]
        for c in copies:
            c.wait()
        pltpu.sync_copy(rows, v_out.at[pl.ds(base, _BPW)])

    return gather_v


def _make_gather_u():
    @functools.partial(
        pl.kernel,
        mesh=_mesh,
        out_type=jax.ShapeDtypeStruct((B, F), jnp.float32),
        scratch_types=[
            pltpu.VMEM((_BPW,), jnp.int32),
            pltpu.VMEM((_BPW, F), jnp.float32),
            pltpu.SemaphoreType.DMA,
        ],
        compiler_params=pltpu.CompilerParams(use_tc_tiling_on_sc=True),
    )
    def gather_u(users_hbm, uemb_hbm, u_out, idx, rows, sem):
        wid = lax.axis_index("s") * _NC + lax.axis_index("c")
        base = wid * _BPW
        pltpu.sync_copy(users_hbm.at[pl.ds(base, _BPW)], idx)

        def group_body(i, _):
            iv = idx[pl.ds(i * _NL, _NL)]
            for j in range(_NL):
                r = iv[j]
                pltpu.async_copy(
                    uemb_hbm.at[r], rows.at[i * _NL + j], sem)
            return 0
        lax.fori_loop(0, _BPW // _NL, group_body, 0)
        pltpu.make_async_copy(
            u_out.at[pl.ds(base, _BPW)], rows, sem).wait()
        pltpu.sync_copy(rows, u_out.at[pl.ds(base, _BPW)])

    return gather_u


_gather_v = _make_gather_v()
_gather_u = _make_gather_u()


def _mlp_body(u_ref, v_ref, w1a_ref, w1b_ref, b1_ref, w2t_ref, b2_ref, o_ref):
    u = u_ref[...]
    v = v_ref[...]
    u = jnp.where(u >= 0, u, 0.01 * u)
    v = jnp.where(v >= 0, v, 0.01 * v)
    h = (
        jnp.dot(u, w1a_ref[...], preferred_element_type=jnp.float32,
                precision=lax.Precision.HIGHEST)
        + jnp.dot(v, w1b_ref[...], preferred_element_type=jnp.float32,
                  precision=lax.Precision.HIGHEST)
        + b1_ref[...]
    )
    h = jnp.where(h >= 0, h, 0.01 * h)
    o = jnp.sum(h * w2t_ref[...], axis=1, keepdims=True) + b2_ref[...]
    o_ref[...] = jnp.maximum(o, 0.0)


_BLK = 2048


@jax.jit
def _mlp(u, v, w1a, w1b, b1, w2t, b2):
    return pl.pallas_call(
        _mlp_body,
        grid=(B // _BLK,),
        in_specs=[
            pl.BlockSpec((_BLK, F), lambda i: (i, 0)),
            pl.BlockSpec((_BLK, F), lambda i: (i, 0)),
            pl.BlockSpec((F, F), lambda i: (0, 0)),
            pl.BlockSpec((F, F), lambda i: (0, 0)),
            pl.BlockSpec((1, F), lambda i: (0, 0)),
            pl.BlockSpec((1, F), lambda i: (0, 0)),
            pl.BlockSpec((1, 1), lambda i: (0, 0)),
        ],
        out_specs=pl.BlockSpec((_BLK, 1), lambda i: (i, 0)),
        out_shape=jax.ShapeDtypeStruct((B, 1), jnp.float32),
    )(u, v, w1a, w1b, b1, w2t, b2)


def kernel(users, items, user_emb, item_emb, W1, b1, W2, b2):
    v = _gather_v(items.astype(jnp.int32), item_emb)
    u = _gather_u(users.astype(jnp.int32), _tru(user_emb.T))
    w1a = W1[:F]
    w1b = W1[F:]
    return _mlp(u, v, w1a, w1b, b1.reshape(1, F), W2.reshape(1, F),
                b2.reshape(1, 1))


# speedup vs baseline: 1.6362x; 1.6362x over previous
import functools

import jax
import jax.numpy as jnp
from jax import lax
from jax.experimental import pallas as pl
from jax.experimental.pallas import tpu as pltpu
from jax.experimental.pallas import tpu_sc as plsc

B = 16384
F = 64

_info = plsc.get_sparse_core_info()
_NC, _NS, _NL = _info.num_cores, _info.num_subcores, _info.num_lanes
_NW = _NC * _NS
_BPW = B // _NW
_CHUNK = 128
_NCHUNK = _BPW // _CHUNK

_mesh = plsc.VectorSubcoreMesh(core_axis_name="c", subcore_axis_name="s")


_TBLK = 8192
_M = 1000000
_TGRID = (_M + _TBLK - 1) // _TBLK


def _tr_body(embt_ref, eye_ref, out_ref):
    x = embt_ref[...]
    hi = x.astype(jnp.bfloat16)
    lo = (x - hi.astype(jnp.float32)).astype(jnp.bfloat16)
    dn = (((0,), (0,)), ((), ()))
    eye = eye_ref[...]
    out_ref[...] = (
        lax.dot_general(hi, eye, dn, preferred_element_type=jnp.float32)
        + lax.dot_general(lo, eye, dn, preferred_element_type=jnp.float32)
    )


@jax.jit
def _tru(embt):
    eye = jnp.eye(F, dtype=jnp.bfloat16)
    return pl.pallas_call(
        _tr_body,
        grid=(_TGRID,),
        in_specs=[
            pl.BlockSpec((F, _TBLK), lambda i: (0, i)),
            pl.BlockSpec((F, F), lambda i: (0, 0)),
        ],
        out_specs=pl.BlockSpec((_TBLK, F), lambda i: (i, 0)),
        out_shape=jax.ShapeDtypeStruct((_M, F), jnp.float32),
    )(embt, eye)


def _make_gather_v():
    @functools.partial(
        pl.kernel,
        mesh=_mesh,
        out_type=jax.ShapeDtypeStruct((B, F), jnp.float32),
        scratch_types=[
            pltpu.VMEM((_BPW,), jnp.int32),
            pltpu.VMEM((_BPW, F), jnp.float32),
            pltpu.SemaphoreType.DMA,
        ],
        compiler_params=pltpu.CompilerParams(use_tc_tiling_on_sc=True),
    )
    def gather_v(items_hbm, vemb_hbm, v_out, idx, rows, sem):
        wid = lax.axis_index("s") * _NC + lax.axis_index("c")
        base = wid * _BPW
        pltpu.sync_copy(items_hbm.at[pl.ds(base, _BPW)], idx)

        def group_body(i, _):
            iv = idx[pl.ds(i * _NL, _NL)]
            for j in range(_NL):
                r = iv[j]
                pltpu.async_copy(
                    vemb_hbm.at[r], rows.at[i * _NL + j], sem)
            return 0
        lax.fori_loop(0, _BPW // _NL, group_body, 0)
        pltpu.make_async_copy(
            v_out.at[pl.ds(base, _BPW)], rows, sem).wait()
        pltpu.sync_copy(rows, v_out.at[pl.ds(base, _BPW)])

    return gather_v


def _make_gather_u():
    @functools.partial(
        pl.kernel,
        mesh=_mesh,
        out_type=jax.ShapeDtypeStruct((B, F), jnp.float32),
        scratch_types=[
            pltpu.VMEM((_BPW,), jnp.int32),
            pltpu.VMEM((_BPW, F), jnp.float32),
            pltpu.SemaphoreType.DMA,
        ],
        compiler_params=pltpu.CompilerParams(use_tc_tiling_on_sc=True),
    )
    def gather_u(users_hbm, uemb_hbm, u_out, idx, rows, sem):
        wid = lax.axis_index("s") * _NC + lax.axis_index("c")
        base = wid * _BPW
        pltpu.sync_copy(users_hbm.at[pl.ds(base, _BPW)], idx)

        def group_body(i, _):
            iv = idx[pl.ds(i * _NL, _NL)]
            for j in range(_NL):
                r = iv[j]
                pltpu.async_copy(
                    uemb_hbm.at[r], rows.at[i * _NL + j], sem)
            return 0
        lax.fori_loop(0, _BPW // _NL, group_body, 0)
        pltpu.make_async_copy(
            u_out.at[pl.ds(base, _BPW)], rows, sem).wait()
        pltpu.sync_copy(rows, u_out.at[pl.ds(base, _BPW)])

    return gather_u


_gather_v = _make_gather_v()
_gather_u = _make_gather_u()


def _mlp_body(u_ref, v_ref, w1a_ref, w1b_ref, b1_ref, w2t_ref, b2_ref, o_ref):
    u = u_ref[...]
    v = v_ref[...]
    u = jnp.where(u >= 0, u, 0.01 * u)
    v = jnp.where(v >= 0, v, 0.01 * v)
    h = (
        jnp.dot(u, w1a_ref[...], preferred_element_type=jnp.float32,
                precision=lax.Precision.HIGHEST)
        + jnp.dot(v, w1b_ref[...], preferred_element_type=jnp.float32,
                  precision=lax.Precision.HIGHEST)
        + b1_ref[...]
    )
    h = jnp.where(h >= 0, h, 0.01 * h)
    o = jnp.sum(h * w2t_ref[...], axis=1, keepdims=True) + b2_ref[...]
    o_ref[...] = jnp.maximum(o, 0.0)


_BLK = 2048


@jax.jit
def _mlp(u, v, w1a, w1b, b1, w2t, b2):
    return pl.pallas_call(
        _mlp_body,
        grid=(B // _BLK,),
        in_specs=[
            pl.BlockSpec((_BLK, F), lambda i: (i, 0)),
            pl.BlockSpec((_BLK, F), lambda i: (i, 0)),
            pl.BlockSpec((F, F), lambda i: (0, 0)),
            pl.BlockSpec((F, F), lambda i: (0, 0)),
            pl.BlockSpec((1, F), lambda i: (0, 0)),
            pl.BlockSpec((1, F), lambda i: (0, 0)),
            pl.BlockSpec((1, 1), lambda i: (0, 0)),
        ],
        out_specs=pl.BlockSpec((_BLK, 1), lambda i: (i, 0)),
        out_shape=jax.ShapeDtypeStruct((B, 1), jnp.float32),
    )(u, v, w1a, w1b, b1, w2t, b2)


def kernel(users, items, user_emb, item_emb, W1, b1, W2, b2):
    v = _gather_v(items.astype(jnp.int32), item_emb)
    u = _gather_u(users.astype(jnp.int32), _tru(user_emb.T))
    w1a = W1[:F]
    w1b = W1[F:]
    return _mlp(u, v, w1a, w1b, b1.reshape(1, F), W2.reshape(1, F),
                b2.reshape(1, 1))


# speedup vs baseline: 2.1815x; 1.3333x over previous
import functools

import jax
import jax.numpy as jnp
from jax import lax
from jax.experimental import pallas as pl
from jax.experimental.pallas import tpu as pltpu
from jax.experimental.pallas import tpu_sc as plsc

B = 16384
F = 64
_ROWS_PER_TILE = 8

_info = plsc.get_sparse_core_info()
_NC, _NS, _NL = _info.num_cores, _info.num_subcores, _info.num_lanes
_NW = _NC * _NS
_BPW = B // _NW
_CHUNK = 64
_NCHUNK = _BPW // _CHUNK


def _make_gather():
    mesh = plsc.VectorSubcoreMesh(core_axis_name="c", subcore_axis_name="s")

    @functools.partial(
        pl.kernel,
        mesh=mesh,
        out_type=[
            jax.ShapeDtypeStruct((B, F), jnp.float32),
            jax.ShapeDtypeStruct((B, F), jnp.float32),
        ],
        scratch_types=[
            pltpu.VMEM((_BPW,), jnp.int32),
            pltpu.VMEM((_CHUNK, F), jnp.float32),
            pltpu.SemaphoreType.DMA,
        ],
    )
    def gather_k(users_hbm, items_hbm, uemb_hbm, iemb_hbm, u_out, v_out,
                 idx, ext, sem):
        wid = lax.axis_index("s") * _NC + lax.axis_index("c")
        base = wid * _BPW

        def one_table(idx_hbm, emb_hbm, out_hbm):
            pltpu.sync_copy(idx_hbm.at[pl.ds(base, _BPW)], idx)

            def chunk_body(c, _):
                def group_body(i, _):
                    iv = idx[pl.ds(c * _CHUNK + i * _NL, _NL)]
                    tvec = lax.shift_right_logical(iv, 3)
                    svec = lax.rem(iv, jnp.int32(_ROWS_PER_TILE))
                    for j in range(_NL):
                        t = tvec[j]
                        s = svec[j]
                        pltpu.async_copy(
                            emb_hbm.at[t, s], ext.at[i * _NL + j], sem)
                    return 0
                lax.fori_loop(0, _CHUNK // _NL, group_body, 0)
                pltpu.make_async_copy(
                    out_hbm.at[pl.ds(base, _CHUNK)], ext, sem).wait()
                pltpu.sync_copy(
                    ext, out_hbm.at[pl.ds(base + c * _CHUNK, _CHUNK)])
                return 0
            lax.fori_loop(0, _NCHUNK, chunk_body, 0)

        one_table(users_hbm, uemb_hbm, u_out)
        one_table(items_hbm, iemb_hbm, v_out)

    return gather_k


_gather = _make_gather()


def _mlp_body(u_ref, v_ref, w1a_ref, w1b_ref, b1_ref, w2t_ref, b2_ref, o_ref):
    u = u_ref[...]
    v = v_ref[...]
    u = jnp.where(u >= 0, u, 0.01 * u)
    v = jnp.where(v >= 0, v, 0.01 * v)
    h = (
        jnp.dot(u, w1a_ref[...], preferred_element_type=jnp.float32,
                precision=lax.Precision.HIGHEST)
        + jnp.dot(v, w1b_ref[...], preferred_element_type=jnp.float32,
                  precision=lax.Precision.HIGHEST)
        + b1_ref[...]
    )
    h = jnp.where(h >= 0, h, 0.01 * h)
    o = jnp.sum(h * w2t_ref[...], axis=1, keepdims=True) + b2_ref[...]
    o_ref[...] = jnp.maximum(o, 0.0)


_BLK = 2048


@jax.jit
def _mlp(u, v, w1a, w1b, b1, w2t, b2):
    return pl.pallas_call(
        _mlp_body,
        grid=(B // _BLK,),
        in_specs=[
            pl.BlockSpec((_BLK, F), lambda i: (i, 0)),
            pl.BlockSpec((_BLK, F), lambda i: (i, 0)),
            pl.BlockSpec((F, F), lambda i: (0, 0)),
            pl.BlockSpec((F, F), lambda i: (0, 0)),
            pl.BlockSpec((1, F), lambda i: (0, 0)),
            pl.BlockSpec((1, F), lambda i: (0, 0)),
            pl.BlockSpec((1, 1), lambda i: (0, 0)),
        ],
        out_specs=pl.BlockSpec((_BLK, 1), lambda i: (i, 0)),
        out_shape=jax.ShapeDtypeStruct((B, 1), jnp.float32),
    )(u, v, w1a, w1b, b1, w2t, b2)


def kernel(users, items, user_emb, item_emb, W1, b1, W2, b2):
    M = user_emb.shape[0]
    N = item_emb.shape[0]
    uemb3 = user_emb.reshape(M // _ROWS_PER_TILE, _ROWS_PER_TILE, F)
    iemb3 = item_emb.reshape(N // _ROWS_PER_TILE, _ROWS_PER_TILE, F)
    u, v = _gather(users.astype(jnp.int32), items.astype(jnp.int32),
                   uemb3, iemb3)
    w1a = W1[:F]
    w1b = W1[F:]
    return _mlp(u, v, w1a, w1b, b1.reshape(1, F), W2.reshape(1, F),
                b2.reshape(1, 1))


# speedup vs baseline: 2.2130x; 1.0144x over previous
import functools

import jax
import jax.numpy as jnp
from jax import lax
from jax.experimental import pallas as pl
from jax.experimental.pallas import tpu as pltpu
from jax.experimental.pallas import tpu_sc as plsc

B = 16384
F = 64
_ROWS_PER_TILE = 8

_info = plsc.get_sparse_core_info()
_NC, _NS, _NL = _info.num_cores, _info.num_subcores, _info.num_lanes
_NW = _NC * _NS
_BPW = B // _NW
_CHUNK = 64
_NCHUNK = _BPW // _CHUNK


def _make_gather():
    mesh = plsc.VectorSubcoreMesh(core_axis_name="c", subcore_axis_name="s")

    @functools.partial(
        pl.kernel,
        mesh=mesh,
        out_type=[
            jax.ShapeDtypeStruct((B, F), jnp.float32),
            jax.ShapeDtypeStruct((B, F), jnp.float32),
        ],
        scratch_types=[
            pltpu.VMEM((_BPW,), jnp.int32),
            pltpu.VMEM((_BPW,), jnp.int32),
            pltpu.VMEM((_CHUNK, F), jnp.float32),
            pltpu.VMEM((_CHUNK, F), jnp.float32),
            pltpu.SemaphoreType.DMA,
            pltpu.SemaphoreType.DMA,
        ],
    )
    def gather_k(users_hbm, items_hbm, uemb_hbm, iemb_hbm, u_out, v_out,
                 idxu, idxv, extu, extv, semu, semv):
        wid = lax.axis_index("s") * _NC + lax.axis_index("c")
        base = wid * _BPW
        pltpu.sync_copy(users_hbm.at[pl.ds(base, _BPW)], idxu)
        pltpu.sync_copy(items_hbm.at[pl.ds(base, _BPW)], idxv)

        def chunk_body(c, _):
            def group_body(i, _):
                ivu = idxu[pl.ds(c * _CHUNK + i * _NL, _NL)]
                ivv = idxv[pl.ds(c * _CHUNK + i * _NL, _NL)]
                tvu = lax.shift_right_logical(ivu, 3)
                svu = lax.rem(ivu, jnp.int32(_ROWS_PER_TILE))
                tvv = lax.shift_right_logical(ivv, 3)
                svv = lax.rem(ivv, jnp.int32(_ROWS_PER_TILE))
                for j in range(_NL):
                    pltpu.async_copy(
                        uemb_hbm.at[tvu[j], svu[j]],
                        extu.at[i * _NL + j], semu)
                    pltpu.async_copy(
                        iemb_hbm.at[tvv[j], svv[j]],
                        extv.at[i * _NL + j], semv)
                return 0
            lax.fori_loop(0, _CHUNK // _NL, group_body, 0)
            pltpu.make_async_copy(
                u_out.at[pl.ds(base, _CHUNK)], extu, semu).wait()
            pltpu.sync_copy(
                extu, u_out.at[pl.ds(base + c * _CHUNK, _CHUNK)])
            pltpu.make_async_copy(
                v_out.at[pl.ds(base, _CHUNK)], extv, semv).wait()
            pltpu.sync_copy(
                extv, v_out.at[pl.ds(base + c * _CHUNK, _CHUNK)])
            return 0
        lax.fori_loop(0, _NCHUNK, chunk_body, 0)

    return gather_k


_gather = _make_gather()


def _mlp_body(u_ref, v_ref, w1a_ref, w1b_ref, b1_ref, w2t_ref, b2_ref, o_ref):
    u = u_ref[...]
    v = v_ref[...]
    u = jnp.where(u >= 0, u, 0.01 * u)
    v = jnp.where(v >= 0, v, 0.01 * v)
    h = (
        jnp.dot(u, w1a_ref[...], preferred_element_type=jnp.float32,
                precision=lax.Precision.HIGHEST)
        + jnp.dot(v, w1b_ref[...], preferred_element_type=jnp.float32,
                  precision=lax.Precision.HIGHEST)
        + b1_ref[...]
    )
    h = jnp.where(h >= 0, h, 0.01 * h)
    o = jnp.sum(h * w2t_ref[...], axis=1, keepdims=True) + b2_ref[...]
    o_ref[...] = jnp.maximum(o, 0.0)


_BLK = 2048


@jax.jit
def _mlp(u, v, w1a, w1b, b1, w2t, b2):
    return pl.pallas_call(
        _mlp_body,
        grid=(B // _BLK,),
        in_specs=[
            pl.BlockSpec((_BLK, F), lambda i: (i, 0)),
            pl.BlockSpec((_BLK, F), lambda i: (i, 0)),
            pl.BlockSpec((F, F), lambda i: (0, 0)),
            pl.BlockSpec((F, F), lambda i: (0, 0)),
            pl.BlockSpec((1, F), lambda i: (0, 0)),
            pl.BlockSpec((1, F), lambda i: (0, 0)),
            pl.BlockSpec((1, 1), lambda i: (0, 0)),
        ],
        out_specs=pl.BlockSpec((_BLK, 1), lambda i: (i, 0)),
        out_shape=jax.ShapeDtypeStruct((B, 1), jnp.float32),
    )(u, v, w1a, w1b, b1, w2t, b2)


def kernel(users, items, user_emb, item_emb, W1, b1, W2, b2):
    M = user_emb.shape[0]
    N = item_emb.shape[0]
    uemb3 = user_emb.reshape(M // _ROWS_PER_TILE, _ROWS_PER_TILE, F)
    iemb3 = item_emb.reshape(N // _ROWS_PER_TILE, _ROWS_PER_TILE, F)
    u, v = _gather(users.astype(jnp.int32), items.astype(jnp.int32),
                   uemb3, iemb3)
    w1a = W1[:F]
    w1b = W1[F:]
    return _mlp(u, v, w1a, w1b, b1.reshape(1, F), W2.reshape(1, F),
                b2.reshape(1, 1))


# speedup vs baseline: 2.2278x; 1.0067x over previous
import functools

import jax
import jax.numpy as jnp
from jax import lax
from jax.experimental import pallas as pl
from jax.experimental.pallas import tpu as pltpu
from jax.experimental.pallas import tpu_sc as plsc

B = 16384
F = 64
_ROWS_PER_TILE = 8

_info = plsc.get_sparse_core_info()
_NC, _NS, _NL = _info.num_cores, _info.num_subcores, _info.num_lanes
_NW = _NC * _NS
_BPW = B // _NW
_CHUNK = 128
_NCHUNK = _BPW // _CHUNK


def _make_gather():
    mesh = plsc.VectorSubcoreMesh(core_axis_name="c", subcore_axis_name="s")

    @functools.partial(
        pl.kernel,
        mesh=mesh,
        out_type=[
            jax.ShapeDtypeStruct((B, F), jnp.float32),
            jax.ShapeDtypeStruct((B, F), jnp.float32),
        ],
        scratch_types=[
            pltpu.VMEM((_BPW,), jnp.int32),
            pltpu.VMEM((_BPW,), jnp.int32),
            pltpu.VMEM((_CHUNK, F), jnp.float32),
            pltpu.VMEM((_CHUNK, F), jnp.float32),
            pltpu.SemaphoreType.DMA,
            pltpu.SemaphoreType.DMA,
        ],
    )
    def gather_k(users_hbm, items_hbm, uemb_hbm, iemb_hbm, u_out, v_out,
                 idxu, idxv, extu, extv, semu, semv):
        wid = lax.axis_index("s") * _NC + lax.axis_index("c")
        base = wid * _BPW
        pltpu.sync_copy(users_hbm.at[pl.ds(base, _BPW)], idxu)
        pltpu.sync_copy(items_hbm.at[pl.ds(base, _BPW)], idxv)

        def chunk_body(c, _):
            def group_body(i, _):
                ivu = idxu[pl.ds(c * _CHUNK + i * _NL, _NL)]
                ivv = idxv[pl.ds(c * _CHUNK + i * _NL, _NL)]
                tvu = lax.shift_right_logical(ivu, 3)
                svu = lax.rem(ivu, jnp.int32(_ROWS_PER_TILE))
                tvv = lax.shift_right_logical(ivv, 3)
                svv = lax.rem(ivv, jnp.int32(_ROWS_PER_TILE))
                for j in range(_NL):
                    pltpu.async_copy(
                        uemb_hbm.at[tvu[j], svu[j]],
                        extu.at[i * _NL + j], semu)
                    pltpu.async_copy(
                        iemb_hbm.at[tvv[j], svv[j]],
                        extv.at[i * _NL + j], semv)
                return 0
            lax.fori_loop(0, _CHUNK // _NL, group_body, 0)
            pltpu.make_async_copy(
                u_out.at[pl.ds(base, _CHUNK)], extu, semu).wait()
            pltpu.sync_copy(
                extu, u_out.at[pl.ds(base + c * _CHUNK, _CHUNK)])
            pltpu.make_async_copy(
                v_out.at[pl.ds(base, _CHUNK)], extv, semv).wait()
            pltpu.sync_copy(
                extv, v_out.at[pl.ds(base + c * _CHUNK, _CHUNK)])
            return 0
        lax.fori_loop(0, _NCHUNK, chunk_body, 0)

    return gather_k


_gather = _make_gather()


def _mlp_body(u_ref, v_ref, w1a_ref, w1b_ref, b1_ref, w2t_ref, b2_ref, o_ref):
    u = u_ref[...]
    v = v_ref[...]
    u = jnp.where(u >= 0, u, 0.01 * u)
    v = jnp.where(v >= 0, v, 0.01 * v)
    h = (
        jnp.dot(u, w1a_ref[...], preferred_element_type=jnp.float32,
                precision=lax.Precision.HIGHEST)
        + jnp.dot(v, w1b_ref[...], preferred_element_type=jnp.float32,
                  precision=lax.Precision.HIGHEST)
        + b1_ref[...]
    )
    h = jnp.where(h >= 0, h, 0.01 * h)
    o = jnp.sum(h * w2t_ref[...], axis=1, keepdims=True) + b2_ref[...]
    o_ref[...] = jnp.maximum(o, 0.0)


_BLK = 2048


@jax.jit
def _mlp(u, v, w1a, w1b, b1, w2t, b2):
    return pl.pallas_call(
        _mlp_body,
        grid=(B // _BLK,),
        in_specs=[
            pl.BlockSpec((_BLK, F), lambda i: (i, 0)),
            pl.BlockSpec((_BLK, F), lambda i: (i, 0)),
            pl.BlockSpec((F, F), lambda i: (0, 0)),
            pl.BlockSpec((F, F), lambda i: (0, 0)),
            pl.BlockSpec((1, F), lambda i: (0, 0)),
            pl.BlockSpec((1, F), lambda i: (0, 0)),
            pl.BlockSpec((1, 1), lambda i: (0, 0)),
        ],
        out_specs=pl.BlockSpec((_BLK, 1), lambda i: (i, 0)),
        out_shape=jax.ShapeDtypeStruct((B, 1), jnp.float32),
    )(u, v, w1a, w1b, b1, w2t, b2)


def kernel(users, items, user_emb, item_emb, W1, b1, W2, b2):
    M = user_emb.shape[0]
    N = item_emb.shape[0]
    uemb3 = user_emb.reshape(M // _ROWS_PER_TILE, _ROWS_PER_TILE, F)
    iemb3 = item_emb.reshape(N // _ROWS_PER_TILE, _ROWS_PER_TILE, F)
    u, v = _gather(users.astype(jnp.int32), items.astype(jnp.int32),
                   uemb3, iemb3)
    w1a = W1[:F]
    w1b = W1[F:]
    return _mlp(u, v, w1a, w1b, b1.reshape(1, F), W2.reshape(1, F),
                b2.reshape(1, 1))


# speedup vs baseline: 2.2364x; 1.0039x over previous
import functools

import jax
import jax.numpy as jnp
from jax import lax
from jax.experimental import pallas as pl
from jax.experimental.pallas import tpu as pltpu
from jax.experimental.pallas import tpu_sc as plsc

B = 16384
F = 64
_ROWS_PER_TILE = 8

_info = plsc.get_sparse_core_info()
_NC, _NS, _NL = _info.num_cores, _info.num_subcores, _info.num_lanes
_NW = _NC * _NS
_BPW = B // _NW
_CHUNK = 256
_NCHUNK = _BPW // _CHUNK


def _make_gather():
    mesh = plsc.VectorSubcoreMesh(core_axis_name="c", subcore_axis_name="s")

    @functools.partial(
        pl.kernel,
        mesh=mesh,
        out_type=[
            jax.ShapeDtypeStruct((B, F), jnp.float32),
            jax.ShapeDtypeStruct((B, F), jnp.float32),
        ],
        scratch_types=[
            pltpu.VMEM((_BPW,), jnp.int32),
            pltpu.VMEM((_BPW,), jnp.int32),
            pltpu.VMEM((_CHUNK, F), jnp.float32),
            pltpu.VMEM((_CHUNK, F), jnp.float32),
            pltpu.SemaphoreType.DMA,
            pltpu.SemaphoreType.DMA,
        ],
    )
    def gather_k(users_hbm, items_hbm, uemb_hbm, iemb_hbm, u_out, v_out,
                 idxu, idxv, extu, extv, semu, semv):
        wid = lax.axis_index("s") * _NC + lax.axis_index("c")
        base = wid * _BPW
        pltpu.sync_copy(users_hbm.at[pl.ds(base, _BPW)], idxu)
        pltpu.sync_copy(items_hbm.at[pl.ds(base, _BPW)], idxv)

        def chunk_body(c, _):
            def group_body(i, _):
                ivu = idxu[pl.ds(c * _CHUNK + i * _NL, _NL)]
                ivv = idxv[pl.ds(c * _CHUNK + i * _NL, _NL)]
                tvu = lax.shift_right_logical(ivu, 3)
                svu = lax.rem(ivu, jnp.int32(_ROWS_PER_TILE))
                tvv = lax.shift_right_logical(ivv, 3)
                svv = lax.rem(ivv, jnp.int32(_ROWS_PER_TILE))
                for j in range(_NL):
                    pltpu.async_copy(
                        uemb_hbm.at[tvu[j], svu[j]],
                        extu.at[i * _NL + j], semu)
                    pltpu.async_copy(
                        iemb_hbm.at[tvv[j], svv[j]],
                        extv.at[i * _NL + j], semv)
                return 0
            lax.fori_loop(0, _CHUNK // _NL, group_body, 0)
            pltpu.make_async_copy(
                u_out.at[pl.ds(base, _CHUNK)], extu, semu).wait()
            pltpu.sync_copy(
                extu, u_out.at[pl.ds(base + c * _CHUNK, _CHUNK)])
            pltpu.make_async_copy(
                v_out.at[pl.ds(base, _CHUNK)], extv, semv).wait()
            pltpu.sync_copy(
                extv, v_out.at[pl.ds(base + c * _CHUNK, _CHUNK)])
            return 0
        lax.fori_loop(0, _NCHUNK, chunk_body, 0)

    return gather_k


_gather = _make_gather()


def _mlp_body(u_ref, v_ref, w1a_ref, w1b_ref, b1_ref, w2t_ref, b2_ref, o_ref):
    u = u_ref[...]
    v = v_ref[...]
    u = jnp.where(u >= 0, u, 0.01 * u)
    v = jnp.where(v >= 0, v, 0.01 * v)
    h = (
        jnp.dot(u, w1a_ref[...], preferred_element_type=jnp.float32,
                precision=lax.Precision.HIGHEST)
        + jnp.dot(v, w1b_ref[...], preferred_element_type=jnp.float32,
                  precision=lax.Precision.HIGHEST)
        + b1_ref[...]
    )
    h = jnp.where(h >= 0, h, 0.01 * h)
    o = jnp.sum(h * w2t_ref[...], axis=1, keepdims=True) + b2_ref[...]
    o_ref[...] = jnp.maximum(o, 0.0)


_BLK = 2048


@jax.jit
def _mlp(u, v, w1a, w1b, b1, w2t, b2):
    return pl.pallas_call(
        _mlp_body,
        grid=(B // _BLK,),
        in_specs=[
            pl.BlockSpec((_BLK, F), lambda i: (i, 0)),
            pl.BlockSpec((_BLK, F), lambda i: (i, 0)),
            pl.BlockSpec((F, F), lambda i: (0, 0)),
            pl.BlockSpec((F, F), lambda i: (0, 0)),
            pl.BlockSpec((1, F), lambda i: (0, 0)),
            pl.BlockSpec((1, F), lambda i: (0, 0)),
            pl.BlockSpec((1, 1), lambda i: (0, 0)),
        ],
        out_specs=pl.BlockSpec((_BLK, 1), lambda i: (i, 0)),
        out_shape=jax.ShapeDtypeStruct((B, 1), jnp.float32),
    )(u, v, w1a, w1b, b1, w2t, b2)


def kernel(users, items, user_emb, item_emb, W1, b1, W2, b2):
    M = user_emb.shape[0]
    N = item_emb.shape[0]
    uemb3 = user_emb.reshape(M // _ROWS_PER_TILE, _ROWS_PER_TILE, F)
    iemb3 = item_emb.reshape(N // _ROWS_PER_TILE, _ROWS_PER_TILE, F)
    u, v = _gather(users.astype(jnp.int32), items.astype(jnp.int32),
                   uemb3, iemb3)
    w1a = W1[:F]
    w1b = W1[F:]
    return _mlp(u, v, w1a, w1b, b1.reshape(1, F), W2.reshape(1, F),
                b2.reshape(1, 1))
